# Initial kernel scaffold; baseline (speedup 1.0000x reference)
import jax
import jax.numpy as jnp
from jax.experimental import pallas as pl


def kernel(x, t, mask, W):
    raise NotImplementedError("write your pallas kernel here")


# speedup vs baseline: 5.1835x; 5.1835x over previous
import jax
import jax.numpy as jnp
from jax.experimental import pallas as pl

B, N, DX, DT, DOUT = 8, 2048, 4, 128, 256


def _body(t_ref, mask_ref, w_ref, out_ref):
    m = mask_ref[...]
    msq = m * m
    tsum = jnp.sum(t_ref[...] * msq, axis=1)
    denom = jnp.maximum(jnp.sum(m, axis=1), 1.0)
    pooled_t = tsum / denom
    pooled_id = msq[:, :, 0] / denom
    w1 = w_ref[:DT, :]
    w2 = w_ref[DT:, :]
    out_ref[...] = (
        jnp.dot(pooled_t, w1, preferred_element_type=jnp.float32)
        + jnp.dot(pooled_id, w2, preferred_element_type=jnp.float32)
    )


def kernel(x, t, mask, W):
    del x
    return pl.pallas_call(
        _body,
        out_shape=jax.ShapeDtypeStruct((B, DOUT), jnp.float32),
    )(t, mask, W)


# speedup vs baseline: 5.8624x; 1.1310x over previous
import jax
import jax.numpy as jnp
from jax.experimental import pallas as pl
from jax.experimental.pallas import tpu as pltpu

B, N, DX, DT, DOUT = 8, 2048, 4, 128, 256
C = 256
STEPS = N // C


def _body(t_ref, mask_ref, w1_ref, w2_ref, out_ref, tsum_ref, idacc_ref, msum_ref):
    i = pl.program_id(0)

    @pl.when(i == 0)
    def _init():
        tsum_ref[...] = jnp.zeros_like(tsum_ref)
        idacc_ref[...] = jnp.zeros_like(idacc_ref)
        msum_ref[...] = jnp.zeros_like(msum_ref)

    m = mask_ref[...]
    msq = m * m
    tsum_ref[...] += jnp.sum(t_ref[...] * msq[:, :, None], axis=1)
    idacc_ref[...] += jnp.dot(msq, w2_ref[...],
                              preferred_element_type=jnp.float32)
    msum_ref[...] += jnp.broadcast_to(
        jnp.sum(m, axis=1, keepdims=True), msum_ref.shape)

    @pl.when(i == STEPS - 1)
    def _finish():
        denom = jnp.maximum(msum_ref[:, :1], 1.0)
        out_ref[...] = (
            jnp.dot(tsum_ref[...] / denom, w1_ref[...],
                    preferred_element_type=jnp.float32)
            + idacc_ref[...] / denom
        )


def kernel(x, t, mask, W):
    del x
    mask2d = jnp.squeeze(mask, -1)
    w1 = W[:DT]
    w2 = W[DT:]
    return pl.pallas_call(
        _body,
        grid=(STEPS,),
        in_specs=[
            pl.BlockSpec((B, C, DT), lambda i: (0, i, 0)),
            pl.BlockSpec((B, C), lambda i: (0, i)),
            pl.BlockSpec((DT, DOUT), lambda i: (0, 0)),
            pl.BlockSpec((C, DOUT), lambda i: (i, 0)),
        ],
        out_specs=pl.BlockSpec((B, DOUT), lambda i: (0, 0)),
        out_shape=jax.ShapeDtypeStruct((B, DOUT), jnp.float32),
        scratch_shapes=[
            pltpu.VMEM((B, DT), jnp.float32),
            pltpu.VMEM((B, DOUT), jnp.float32),
            pltpu.VMEM((B, 128), jnp.float32),
        ],
    )(t, mask2d, w1, w2)


# speedup vs baseline: 6.5789x; 1.1222x over previous
import jax
import jax.numpy as jnp
from jax.experimental import pallas as pl

B, N, DX, DT, DOUT = 8, 2048, 4, 128, 256


def _body(t_ref, mask_ref, w1_ref, w2_ref, out_ref):
    i = pl.program_id(0)
    m = mask_ref[0]
    msq = m * m
    denom = jnp.maximum(jnp.sum(m, axis=1, keepdims=True), 1.0)
    tvec = jnp.dot(msq, t_ref[0], preferred_element_type=jnp.float32)
    g = (
        jnp.dot(tvec / denom, w1_ref[...], preferred_element_type=jnp.float32)
        + jnp.dot(msq / denom, w2_ref[...], preferred_element_type=jnp.float32)
    )
    out_ref[pl.ds(i, 1), :] = g


def kernel(x, t, mask, W):
    del x
    mask3d = jnp.reshape(mask, (B, 1, N))
    w1 = W[:DT]
    w2 = W[DT:]
    return pl.pallas_call(
        _body,
        grid=(B,),
        in_specs=[
            pl.BlockSpec((1, N, DT), lambda i: (i, 0, 0)),
            pl.BlockSpec((1, 1, N), lambda i: (i, 0, 0)),
            pl.BlockSpec((DT, DOUT), lambda i: (0, 0)),
            pl.BlockSpec((N, DOUT), lambda i: (0, 0)),
        ],
        out_specs=pl.BlockSpec((B, DOUT), lambda i: (0, 0)),
        out_shape=jax.ShapeDtypeStruct((B, DOUT), jnp.float32),
    )(t, mask3d, w1, w2)
